# baseline (speedup 1.0000x reference)
import functools

import jax
import jax.numpy as jnp
from jax import lax
from jax.experimental import pallas as pl
from jax.experimental.pallas import tpu as pltpu
from jax.experimental.pallas import tpu_sc as plsc

_HID = 128
_K = 48
_VOCAB = 21
_L = 512
_B = 4
_NB = _B * _L
_ROWS = _NB * _K
_MU0 = 2.0
_MUSTEP = 20.0 / 15.0
_INVSIG = 16.0 / 20.0
_BIG = 3e38


def _prep_body(ca_ref, cat_ref, s_ref, mp_ref, mw_ref, ws_ref, wm1_ref,
               bm1_ref, dn_ref, ei_ref, wt_ref, g_ref, a_ref):
    b = pl.program_id(0)
    ca = ca_ref[0]
    cat = cat_ref[0]
    ca = jnp.where(jnp.isnan(ca), 0.0, ca)
    cat = jnp.where(jnp.isnan(cat), 0.0, cat)

    acc = jnp.zeros((_L, _L), jnp.float32)
    for c in range(3):
        d = ca[:, c:c + 1] - cat[c:c + 1, :]
        acc = acc + d * d
    dist = jnp.sqrt(acc + 1e-6)

    subl = lax.broadcasted_iota(jnp.int32, (_L, _L), 0)
    keys = (lax.bitcast_convert_type(dist, jnp.int32) & ~7) | (subl & 7)
    giota = lax.broadcasted_iota(jnp.int32, (_L // 8, _L), 0)
    dn_rows = []
    ei_rows = []
    for _ in range(_K):
        gkey = jnp.min(keys.reshape(_L // 8, 8, _L), axis=1)
        mnk = jnp.min(gkey, axis=0, keepdims=True)
        g = jnp.min(jnp.where(gkey == mnk, giota, _L),
                    axis=0, keepdims=True)
        sel = g * 8 + (mnk & 7)
        dn_rows.append(lax.bitcast_convert_type(mnk & ~7, jnp.float32))
        ei_rows.append(sel)
        keys = jnp.where(subl == sel, jnp.int32(0x7FFFFFFF), keys)
    dn = jnp.concatenate(dn_rows, axis=0)
    ei = jnp.concatenate(ei_rows, axis=0)
    dn_ref[:, 0, 0, :] = dn
    ei_ref[:, 0, 0, :] = ei + b * _L

    s = s_ref[0]
    s = s.reshape(_L, 1)
    row = lax.broadcasted_iota(jnp.int32, (_L, 1), 0)
    for m in range(2):
        pos = mp_ref[b, m]
        aa = mw_ref[b, m]
        s = jnp.where(row == pos, aa, s)
    wt = jnp.zeros((_L, _HID), jnp.float32)
    for v in range(_VOCAB):
        wt = wt + jnp.where(s == v, 1.0, 0.0) * ws_ref[v:v + 1, :]
    wt_ref[0] = wt

    w1a = wm1_ref[0, :_HID, :]
    w1b = wm1_ref[0, _HID:2 * _HID, :]
    g_ref[0] = jnp.dot(wt, w1b, preferred_element_type=jnp.float32)
    a_ref[0] = (jnp.dot(wt, w1a, preferred_element_type=jnp.float32)
                + bm1_ref[0:1, :])


def _run_prep(ca, cat, s3, mp, mw, ws, wm1, bm1):
    f32 = jnp.float32
    return pl.pallas_call(
        _prep_body,
        grid=(_B,),
        in_specs=[
            pl.BlockSpec((1, _L, 3), lambda b: (b, 0, 0)),
            pl.BlockSpec((1, 3, _L), lambda b: (b, 0, 0)),
            pl.BlockSpec((1, 1, _L), lambda b: (b, 0, 0)),
            pl.BlockSpec(memory_space=pltpu.SMEM),
            pl.BlockSpec(memory_space=pltpu.SMEM),
            pl.BlockSpec((_VOCAB, _HID), lambda b: (0, 0)),
            pl.BlockSpec((3, 3 * _HID, _HID), lambda b: (0, 0, 0)),
            pl.BlockSpec((3, _HID), lambda b: (0, 0)),
        ],
        out_specs=[
            pl.BlockSpec((_K, 1, 1, _L), lambda b: (0, b, 0, 0)),
            pl.BlockSpec((_K, 1, 1, _L), lambda b: (0, b, 0, 0)),
            pl.BlockSpec((1, _L, _HID), lambda b: (b, 0, 0)),
            pl.BlockSpec((1, _L, _HID), lambda b: (b, 0, 0)),
            pl.BlockSpec((1, _L, _HID), lambda b: (b, 0, 0)),
        ],
        out_shape=[
            jax.ShapeDtypeStruct((_K, _B, 1, _L), f32),
            jax.ShapeDtypeStruct((_K, _B, 1, _L), jnp.int32),
            jax.ShapeDtypeStruct((_B, _L, _HID), f32),
            jax.ShapeDtypeStruct((_B, _L, _HID), f32),
            jax.ShapeDtypeStruct((_B, _L, _HID), f32),
        ],
    )(ca, cat, s3, mp, mw, ws, wm1, bm1)


_NW = 32
_RPW = _ROWS // _NW
_NBUF = 4
_CHUNK = 192
_NCH = _RPW // _CHUNK
_LOOKAHEAD = 3


def _sc_gather_body(table_ref, idx_ref, out_ref, *scratch):
    idx_bufs = scratch[:_NBUF]
    row_bufs = scratch[_NBUF:2 * _NBUF]
    gsems = scratch[2 * _NBUF:3 * _NBUF]
    ssems = scratch[3 * _NBUF:4 * _NBUF]
    shared = scratch[4 * _NBUF]
    sid = lax.axis_index("s")
    wid = sid * 2 + lax.axis_index("c")
    base = wid * _RPW

    pltpu.sync_copy(table_ref.at[pl.ds(sid * (_NB // 16), _NB // 16)],
                    shared.at[pl.ds(sid * (_NB // 16), _NB // 16)])
    plsc.subcore_barrier()

    def start_gather(c):
        b = c % _NBUF
        pltpu.sync_copy(idx_ref.at[pl.ds(base + c * _CHUNK, _CHUNK)],
                        idx_bufs[b])
        return pltpu.async_copy(shared.at[idx_bufs[b]], row_bufs[b],
                                gsems[b])

    gh = [None] * _NBUF
    sh = [None] * _NBUF
    for c in range(min(_LOOKAHEAD, _NCH)):
        gh[c % _NBUF] = start_gather(c)
    for c in range(_NCH):
        b = c % _NBUF
        gh[b].wait()
        sh[b] = pltpu.async_copy(
            row_bufs[b], out_ref.at[pl.ds(base + c * _CHUNK, _CHUNK)],
            ssems[b])
        n = c + _LOOKAHEAD
        if n < _NCH:
            nb = n % _NBUF
            if sh[nb] is not None:
                sh[nb].wait()
            gh[nb] = start_gather(n)
    for h in sh:
        if h is not None:
            h.wait()


@functools.cache
def _make_gather():
    return functools.partial(
        pl.kernel,
        out_type=jax.ShapeDtypeStruct((_ROWS, _HID), jnp.float32),
        mesh=plsc.VectorSubcoreMesh(core_axis_name="c", subcore_axis_name="s"),
        scratch_types=(
            [pltpu.VMEM((_CHUNK,), jnp.int32)] * _NBUF
            + [pltpu.VMEM((_CHUNK, _HID), jnp.float32)] * _NBUF
            + [pltpu.SemaphoreType.DMA] * (2 * _NBUF)
            + [pltpu.VMEM_SHARED((_NB, _HID), jnp.float32)]
        ),
    )(_sc_gather_body)


def _gather_rows(table, idx):
    return _make_gather()(table, idx)


_NBLK = 256
_GBLK = _NBLK * _K


def _comb_body(layer, last, g_ref, dn_ref, a_ref, h_ref, wm1_ref, bm1_ref,
               wm2_ref, bm2_ref, we_ref, *out_refs):
    dn = dn_ref[...]
    mu = _MU0 + lax.broadcasted_iota(
        jnp.int32, (_K, _NBLK, 16), 2).astype(jnp.float32) * _MUSTEP
    t = (dn[:, :, None] - mu) * _INVSIG
    p = jnp.exp(-(t * t)).reshape(_GBLK, 16)
    cmat = jnp.dot(we_ref[...], wm1_ref[layer, 2 * _HID:, :],
                   preferred_element_type=jnp.float32)
    r = jnp.dot(p, cmat, preferred_element_type=jnp.float32)

    a = a_ref[...]
    pre = jnp.maximum(g_ref[...].astype(jnp.float32)
                      + r.reshape(_K, _NBLK, _HID)
                      + a[None, :, :], 0.0)
    m = jnp.mean(pre, axis=0)

    msg = jnp.dot(m, wm2_ref[layer], preferred_element_type=jnp.float32) \
        + bm2_ref[layer:layer + 1, :]
    hm = h_ref[...] + msg
    mean = jnp.mean(hm, axis=1, keepdims=True)
    cen = hm - mean
    var = jnp.mean(cen * cen, axis=1, keepdims=True)
    h_new = cen * lax.rsqrt(var + 1e-5)
    out_refs[0][...] = h_new
    if not last:
        nl = layer + 1
        out_refs[1][...] = jnp.dot(h_new, wm1_ref[nl, _HID:2 * _HID, :],
                                   preferred_element_type=jnp.float32)
        out_refs[2][...] = (jnp.dot(h_new, wm1_ref[nl, :_HID, :],
                                    preferred_element_type=jnp.float32)
                            + bm1_ref[nl:nl + 1, :])


def _run_comb(layer, last, gath, dn_f, a_f, h_f, wm1, bm1, wm2, bm2, we):
    f32 = jnp.float32
    nsteps = _NB // _NBLK
    out_shape = [jax.ShapeDtypeStruct((_NB, _HID), f32)]
    out_specs = [pl.BlockSpec((_NBLK, _HID), lambda i: (i, 0))]
    if not last:
        out_shape += [jax.ShapeDtypeStruct((_NB, _HID), f32)] * 2
        out_specs += [pl.BlockSpec((_NBLK, _HID), lambda i: (i, 0))] * 2
    return pl.pallas_call(
        functools.partial(_comb_body, layer, last),
        grid=(nsteps,),
        in_specs=[
            pl.BlockSpec((_K, _NBLK, _HID), lambda i: (0, i, 0)),
            pl.BlockSpec((_K, _NBLK), lambda i: (0, i)),
            pl.BlockSpec((_NBLK, _HID), lambda i: (i, 0)),
            pl.BlockSpec((_NBLK, _HID), lambda i: (i, 0)),
            pl.BlockSpec((3, 3 * _HID, _HID), lambda i: (0, 0, 0)),
            pl.BlockSpec((3, _HID), lambda i: (0, 0)),
            pl.BlockSpec((3, _HID, _HID), lambda i: (0, 0, 0)),
            pl.BlockSpec((3, _HID), lambda i: (0, 0)),
            pl.BlockSpec((16, _HID), lambda i: (0, 0)),
        ],
        out_specs=out_specs,
        out_shape=out_shape,
    )(gath, dn_f, a_f, h_f, wm1, bm1, wm2, bm2, we)


def _head_body(h1_ref, h2_ref, wt_ref, dn_ref, ei_ref, mp_ref, mw_ref,
               mm_ref, ws_ref, we_ref, lng_ref, lnb_ref, wla_ref, bla_ref,
               wd1_ref, bd1_ref, wd2_ref, bd2_ref, out_ref):
    node = lax.broadcasted_iota(jnp.int32, (_NB, 1), 0)

    def pick_row(ref, flatpos, width):
        mask = node == flatpos
        return jnp.sum(jnp.where(mask, ref[...], 0.0), axis=0,
                       keepdims=True)[:, :width]

    def pick_row_i(ref, flatpos, width):
        mask = node == flatpos
        return jnp.sum(jnp.where(mask, ref[...], 0), axis=0,
                       keepdims=True)[:, :width]

    def embed_of(aa):
        e = jnp.zeros((1, _HID), jnp.float32)
        for v in range(_VOCAB):
            e = e + jnp.where(aa == v, 1.0, 0.0) * ws_ref[v:v + 1, :]
        return e

    fe_rows = []
    for i in range(2):
        for b in range(_B):
            pos_c = mp_ref[b, i]
            pos_o = mp_ref[b, 1 - i]
            flat_c = b * _L + pos_c
            h1r = pick_row(h1_ref, flat_c, _HID)
            h2r = pick_row(h2_ref, flat_c, _HID)
            wtr = pick_row(wt_ref, flat_c, _HID)
            colmask = lax.broadcasted_iota(jnp.int32, (_K, _NB), 1) == flat_c
            dnr = jnp.sum(jnp.where(colmask, dn_ref[...], 0.0), axis=1,
                          keepdims=True)
            eir = jnp.sum(jnp.where(colmask, ei_ref[...], 0), axis=1,
                          keepdims=True)
            match = jnp.where(eir == b * _L + pos_o, 1.0, 0.0)
            mu = _MU0 + lax.broadcasted_iota(
                jnp.int32, (_K, 16), 1).astype(jnp.float32) * _MUSTEP
            t = (dnr - mu) * _INVSIG
            rb = jnp.exp(-(t * t))
            msum = jnp.sum(match * rb, axis=0, keepdims=True)
            edge = jnp.dot(msum, we_ref[...],
                           preferred_element_type=jnp.float32)
            g_seq = wtr - embed_of(mm_ref[b, i])
            fe_rows.append(jnp.concatenate([h1r, h2r, g_seq, edge], axis=1))
    fe = jnp.concatenate(fe_rows, axis=0)

    mean = jnp.mean(fe, axis=1, keepdims=True)
    cen = fe - mean
    var = jnp.mean(cen * cen, axis=1, keepdims=True)
    fe = cen * lax.rsqrt(var + 1e-5) * lng_ref[...] + lnb_ref[...]
    fe = jnp.maximum(jnp.dot(fe, wla_ref[...],
                             preferred_element_type=jnp.float32)
                     + bla_ref[...], 0.0)

    sm_rows = []
    for i in range(2):
        for b in range(_B):
            if i == 0:
                sm_rows.append(jnp.ones((1, 1), jnp.float32))
            else:
                dead = jnp.logical_and(mw_ref[b, 1] == 0, mm_ref[b, 1] == 0)
                sm_rows.append(jnp.where(dead, 0.0, 1.0) * jnp.ones((1, 1), jnp.float32))
    fe = fe * jnp.concatenate(sm_rows, axis=0)

    swapped = jnp.concatenate([fe[4:8], fe[0:4]], axis=0)
    embeds = jnp.concatenate([fe, swapped], axis=1)
    hmid = jnp.maximum(jnp.dot(embeds, wd1_ref[...],
                               preferred_element_type=jnp.float32)
                       + bd1_ref[...], 0.0)
    dd = jnp.dot(hmid, wd2_ref[...],
                 preferred_element_type=jnp.float32) + bd2_ref[...]

    lane441 = lax.broadcasted_iota(jnp.int32, (8, _VOCAB * _VOCAB), 1)
    mcols, wcols = [], []
    for i in range(2):
        for b in range(_B):
            if i == 0:
                mi = mm_ref[b, 0] * _VOCAB + mm_ref[b, 1]
                wi = mw_ref[b, 0] * _VOCAB + mw_ref[b, 1]
            else:
                mi = mm_ref[b, 1] * _VOCAB + mm_ref[b, 0]
                wi = mw_ref[b, 1] * _VOCAB + mw_ref[b, 0]
            mcols.append(jnp.full((1, 1), 0, jnp.int32) + mi)
            wcols.append(jnp.full((1, 1), 0, jnp.int32) + wi)
    mcol = jnp.concatenate(mcols, axis=0)
    wcol = jnp.concatenate(wcols, axis=0)
    mval = jnp.sum(jnp.where(lane441 == mcol, dd, 0.0), axis=1, keepdims=True)
    wval = jnp.sum(jnp.where(lane441 == wcol, dd, 0.0), axis=1, keepdims=True)
    diff = mval - wval
    res = 0.5 * (diff[0:4] + diff[4:8])
    res = jnp.concatenate([res, jnp.zeros((4, 1), jnp.float32)], axis=0)
    lane128 = lax.broadcasted_iota(jnp.int32, (8, _HID), 1)
    out_ref[...] = jnp.where(lane128 == 0, res, 0.0)


def _run_head(h1f, h2f, wtf, dnf, eif, mp, mw, mm, ws, we, lng, lnb, wla,
              bla, wd1, bd1, wd2, bd2):
    vspec = lambda shape: pl.BlockSpec(shape, lambda: tuple(0 for _ in shape))
    return pl.pallas_call(
        _head_body,
        in_specs=[
            vspec((_NB, _HID)), vspec((_NB, _HID)), vspec((_NB, _HID)),
            vspec((_K, _NB)), vspec((_K, _NB)),
            pl.BlockSpec(memory_space=pltpu.SMEM),
            pl.BlockSpec(memory_space=pltpu.SMEM),
            pl.BlockSpec(memory_space=pltpu.SMEM),
            vspec((_VOCAB, _HID)), vspec((16, _HID)),
            vspec((1, 512)), vspec((1, 512)),
            vspec((512, _HID)), vspec((1, _HID)),
            vspec((2 * _HID, _HID)), vspec((1, _HID)),
            vspec((_HID, _VOCAB * _VOCAB)), vspec((1, _VOCAB * _VOCAB)),
        ],
        out_specs=vspec((8, _HID)),
        out_shape=jax.ShapeDtypeStruct((8, _HID), jnp.float32),
    )(h1f, h2f, wtf, dnf, eif, mp, mw, mm, ws, we, lng, lnb, wla, bla,
      wd1, bd1, wd2, bd2)


def kernel(X, S, mask, chain_M, residue_idx, chain_encoding_all,
           mut_positions, mut_wildtype_AAs, mut_mutant_AAs, mut_ddGs,
           atom_mask, W_s, W_e, Wm1, bm1, Wm2, bm2, ln_g, ln_b, W_la, b_la,
           W_d1, b_d1, W_d2, b_d2):
    ca = X[:, :, 1, :]
    cat = jnp.transpose(ca, (0, 2, 1))
    s3 = S.astype(jnp.int32).reshape(_B, 1, _L)
    mp = mut_positions.astype(jnp.int32)
    mw = mut_wildtype_AAs.astype(jnp.int32)
    mm = mut_mutant_AAs.astype(jnp.int32)

    dn, ei, wt, g0, a0 = _run_prep(ca, cat, s3, mp, mw, W_s, Wm1, bm1)

    dn_f = dn.reshape(_K, _NB)
    ei_flat = ei.reshape(_ROWS)
    a_f = a0.reshape(_NB, _HID)
    h_f = wt.reshape(_NB, _HID)
    g_f = g0.reshape(_NB, _HID)

    gath0 = _gather_rows(g_f, ei_flat).reshape(_K, _NB, _HID)
    h1, g1, a1 = _run_comb(0, False, gath0, dn_f, a_f, h_f, Wm1, bm1,
                           Wm2, bm2, W_e)
    gath1 = _gather_rows(g1, ei_flat).reshape(_K, _NB, _HID)
    (h2,) = _run_comb(1, True, gath1, dn_f, a1, h1, Wm1, bm1, Wm2, bm2, W_e)

    res = _run_head(h1, h2, h_f, dn_f, ei.reshape(_K, _NB), mp, mw, mm,
                    W_s, W_e, ln_g.reshape(1, 512), ln_b.reshape(1, 512),
                    W_la, b_la.reshape(1, _HID), W_d1, b_d1.reshape(1, _HID),
                    W_d2, b_d2.reshape(1, _VOCAB * _VOCAB))
    return res[0:4, 0:1]


# speedup vs baseline: 1.5976x; 1.5976x over previous
import functools

import jax
import jax.numpy as jnp
from jax import lax
from jax.experimental import pallas as pl
from jax.experimental.pallas import tpu as pltpu
from jax.experimental.pallas import tpu_sc as plsc

_HID = 128
_K = 48
_VOCAB = 21
_L = 512
_B = 4
_NB = _B * _L
_ROWS = _NB * _K
_MU0 = 2.0
_MUSTEP = 20.0 / 15.0
_INVSIG = 16.0 / 20.0
_BIG = 3e38


def _prep_body(ca_ref, cat_ref, s_ref, mp_ref, mw_ref, ws_ref, wm1_ref,
               bm1_ref, dn_ref, ei_ref, wt_ref, g_ref, a_ref):
    b = pl.program_id(0)
    ca = ca_ref[0]
    cat = cat_ref[0]
    ca = jnp.where(jnp.isnan(ca), 0.0, ca)
    cat = jnp.where(jnp.isnan(cat), 0.0, cat)

    acc = jnp.zeros((_L, _L), jnp.float32)
    for c in range(3):
        d = ca[:, c:c + 1] - cat[c:c + 1, :]
        acc = acc + d * d
    dist = jnp.sqrt(acc + 1e-6)

    subl = lax.broadcasted_iota(jnp.int32, (_L, _L), 0)
    keys = (lax.bitcast_convert_type(dist, jnp.int32) & ~511) | subl
    dn_rows = []
    ei_rows = []
    for _ in range(_K):
        mnk = jnp.min(keys, axis=0, keepdims=True)
        sel = mnk & 511
        dn_rows.append(lax.bitcast_convert_type(mnk & ~511, jnp.float32))
        ei_rows.append(sel)
        keys = jnp.where(subl == sel, jnp.int32(0x7FFFFFFF), keys)
    dn = jnp.concatenate(dn_rows, axis=0)
    ei = jnp.concatenate(ei_rows, axis=0)
    dn_ref[:, 0, 0, :] = dn
    ei_ref[:, 0, 0, :] = ei + b * _L

    s = s_ref[0]
    s = s.reshape(_L, 1)
    row = lax.broadcasted_iota(jnp.int32, (_L, 1), 0)
    for m in range(2):
        pos = mp_ref[b, m]
        aa = mw_ref[b, m]
        s = jnp.where(row == pos, aa, s)
    wt = jnp.zeros((_L, _HID), jnp.float32)
    for v in range(_VOCAB):
        wt = wt + jnp.where(s == v, 1.0, 0.0) * ws_ref[v:v + 1, :]
    wt_ref[0] = wt

    w1a = wm1_ref[0, :_HID, :]
    w1b = wm1_ref[0, _HID:2 * _HID, :]
    g_ref[0] = jnp.dot(wt, w1b, preferred_element_type=jnp.float32)
    a_ref[0] = (jnp.dot(wt, w1a, preferred_element_type=jnp.float32)
                + bm1_ref[0:1, :])


def _run_prep(ca, cat, s3, mp, mw, ws, wm1, bm1):
    f32 = jnp.float32
    return pl.pallas_call(
        _prep_body,
        grid=(_B,),
        in_specs=[
            pl.BlockSpec((1, _L, 3), lambda b: (b, 0, 0)),
            pl.BlockSpec((1, 3, _L), lambda b: (b, 0, 0)),
            pl.BlockSpec((1, 1, _L), lambda b: (b, 0, 0)),
            pl.BlockSpec(memory_space=pltpu.SMEM),
            pl.BlockSpec(memory_space=pltpu.SMEM),
            pl.BlockSpec((_VOCAB, _HID), lambda b: (0, 0)),
            pl.BlockSpec((3, 3 * _HID, _HID), lambda b: (0, 0, 0)),
            pl.BlockSpec((3, _HID), lambda b: (0, 0)),
        ],
        out_specs=[
            pl.BlockSpec((_K, 1, 1, _L), lambda b: (0, b, 0, 0)),
            pl.BlockSpec((_K, 1, 1, _L), lambda b: (0, b, 0, 0)),
            pl.BlockSpec((1, _L, _HID), lambda b: (b, 0, 0)),
            pl.BlockSpec((1, _L, _HID), lambda b: (b, 0, 0)),
            pl.BlockSpec((1, _L, _HID), lambda b: (b, 0, 0)),
        ],
        out_shape=[
            jax.ShapeDtypeStruct((_K, _B, 1, _L), f32),
            jax.ShapeDtypeStruct((_K, _B, 1, _L), jnp.int32),
            jax.ShapeDtypeStruct((_B, _L, _HID), f32),
            jax.ShapeDtypeStruct((_B, _L, _HID), f32),
            jax.ShapeDtypeStruct((_B, _L, _HID), f32),
        ],
    )(ca, cat, s3, mp, mw, ws, wm1, bm1)


_NW = 32
_RPW = _ROWS // _NW
_NBUF = 4
_CHUNK = 192
_NCH = _RPW // _CHUNK
_LOOKAHEAD = 3


def _sc_gather_body(table_ref, idx_ref, out_ref, *scratch):
    idx_bufs = scratch[:_NBUF]
    row_bufs = scratch[_NBUF:2 * _NBUF]
    gsems = scratch[2 * _NBUF:3 * _NBUF]
    ssems = scratch[3 * _NBUF:4 * _NBUF]
    shared = scratch[4 * _NBUF]
    sid = lax.axis_index("s")
    wid = sid * 2 + lax.axis_index("c")
    base = wid * _RPW

    pltpu.sync_copy(table_ref.at[pl.ds(sid * (_NB // 16), _NB // 16)],
                    shared.at[pl.ds(sid * (_NB // 16), _NB // 16)])
    plsc.subcore_barrier()

    def start_gather(c):
        b = c % _NBUF
        pltpu.sync_copy(idx_ref.at[pl.ds(base + c * _CHUNK, _CHUNK)],
                        idx_bufs[b])
        return pltpu.async_copy(shared.at[idx_bufs[b]], row_bufs[b],
                                gsems[b])

    gh = [None] * _NBUF
    sh = [None] * _NBUF
    for c in range(min(_LOOKAHEAD, _NCH)):
        gh[c % _NBUF] = start_gather(c)
    for c in range(_NCH):
        b = c % _NBUF
        gh[b].wait()
        sh[b] = pltpu.async_copy(
            row_bufs[b], out_ref.at[pl.ds(base + c * _CHUNK, _CHUNK)],
            ssems[b])
        n = c + _LOOKAHEAD
        if n < _NCH:
            nb = n % _NBUF
            if sh[nb] is not None:
                sh[nb].wait()
            gh[nb] = start_gather(n)
    for h in sh:
        if h is not None:
            h.wait()


@functools.cache
def _make_gather():
    return functools.partial(
        pl.kernel,
        out_type=jax.ShapeDtypeStruct((_ROWS, _HID), jnp.float32),
        mesh=plsc.VectorSubcoreMesh(core_axis_name="c", subcore_axis_name="s"),
        scratch_types=(
            [pltpu.VMEM((_CHUNK,), jnp.int32)] * _NBUF
            + [pltpu.VMEM((_CHUNK, _HID), jnp.float32)] * _NBUF
            + [pltpu.SemaphoreType.DMA] * (2 * _NBUF)
            + [pltpu.VMEM_SHARED((_NB, _HID), jnp.float32)]
        ),
    )(_sc_gather_body)


def _gather_rows(table, idx):
    return _make_gather()(table, idx)


_NBLK = 256
_GBLK = _NBLK * _K


def _comb_body(layer, last, g_ref, dn_ref, a_ref, h_ref, wm1_ref, bm1_ref,
               wm2_ref, bm2_ref, we_ref, *out_refs):
    dn = dn_ref[...]
    mu = _MU0 + lax.broadcasted_iota(
        jnp.int32, (_K, _NBLK, 16), 2).astype(jnp.float32) * _MUSTEP
    t = (dn[:, :, None] - mu) * _INVSIG
    p = jnp.exp(-(t * t)).reshape(_GBLK, 16)
    cmat = jnp.dot(we_ref[...], wm1_ref[layer, 2 * _HID:, :],
                   preferred_element_type=jnp.float32)
    r = jnp.dot(p, cmat, preferred_element_type=jnp.float32)

    a = a_ref[...]
    pre = jnp.maximum(g_ref[...].astype(jnp.float32)
                      + r.reshape(_K, _NBLK, _HID)
                      + a[None, :, :], 0.0)
    m = jnp.mean(pre, axis=0)

    msg = jnp.dot(m, wm2_ref[layer], preferred_element_type=jnp.float32) \
        + bm2_ref[layer:layer + 1, :]
    hm = h_ref[...] + msg
    mean = jnp.mean(hm, axis=1, keepdims=True)
    cen = hm - mean
    var = jnp.mean(cen * cen, axis=1, keepdims=True)
    h_new = cen * lax.rsqrt(var + 1e-5)
    out_refs[0][...] = h_new
    if not last:
        nl = layer + 1
        out_refs[1][...] = jnp.dot(h_new, wm1_ref[nl, _HID:2 * _HID, :],
                                   preferred_element_type=jnp.float32)
        out_refs[2][...] = (jnp.dot(h_new, wm1_ref[nl, :_HID, :],
                                    preferred_element_type=jnp.float32)
                            + bm1_ref[nl:nl + 1, :])


def _run_comb(layer, last, gath, dn_f, a_f, h_f, wm1, bm1, wm2, bm2, we):
    f32 = jnp.float32
    nsteps = _NB // _NBLK
    out_shape = [jax.ShapeDtypeStruct((_NB, _HID), f32)]
    out_specs = [pl.BlockSpec((_NBLK, _HID), lambda i: (i, 0))]
    if not last:
        out_shape += [jax.ShapeDtypeStruct((_NB, _HID), f32)] * 2
        out_specs += [pl.BlockSpec((_NBLK, _HID), lambda i: (i, 0))] * 2
    return pl.pallas_call(
        functools.partial(_comb_body, layer, last),
        grid=(nsteps,),
        in_specs=[
            pl.BlockSpec((_K, _NBLK, _HID), lambda i: (0, i, 0)),
            pl.BlockSpec((_K, _NBLK), lambda i: (0, i)),
            pl.BlockSpec((_NBLK, _HID), lambda i: (i, 0)),
            pl.BlockSpec((_NBLK, _HID), lambda i: (i, 0)),
            pl.BlockSpec((3, 3 * _HID, _HID), lambda i: (0, 0, 0)),
            pl.BlockSpec((3, _HID), lambda i: (0, 0)),
            pl.BlockSpec((3, _HID, _HID), lambda i: (0, 0, 0)),
            pl.BlockSpec((3, _HID), lambda i: (0, 0)),
            pl.BlockSpec((16, _HID), lambda i: (0, 0)),
        ],
        out_specs=out_specs,
        out_shape=out_shape,
    )(gath, dn_f, a_f, h_f, wm1, bm1, wm2, bm2, we)


def _head_body(h1_ref, h2_ref, wt_ref, dn_ref, ei_ref, mp_ref, mw_ref,
               mm_ref, ws_ref, we_ref, lng_ref, lnb_ref, wla_ref, bla_ref,
               wd1_ref, bd1_ref, wd2_ref, bd2_ref, out_ref):
    node = lax.broadcasted_iota(jnp.int32, (_NB, 1), 0)

    def pick_row(ref, flatpos, width):
        mask = node == flatpos
        return jnp.sum(jnp.where(mask, ref[...], 0.0), axis=0,
                       keepdims=True)[:, :width]

    def pick_row_i(ref, flatpos, width):
        mask = node == flatpos
        return jnp.sum(jnp.where(mask, ref[...], 0), axis=0,
                       keepdims=True)[:, :width]

    def embed_of(aa):
        e = jnp.zeros((1, _HID), jnp.float32)
        for v in range(_VOCAB):
            e = e + jnp.where(aa == v, 1.0, 0.0) * ws_ref[v:v + 1, :]
        return e

    fe_rows = []
    for i in range(2):
        for b in range(_B):
            pos_c = mp_ref[b, i]
            pos_o = mp_ref[b, 1 - i]
            flat_c = b * _L + pos_c
            h1r = pick_row(h1_ref, flat_c, _HID)
            h2r = pick_row(h2_ref, flat_c, _HID)
            wtr = pick_row(wt_ref, flat_c, _HID)
            colmask = lax.broadcasted_iota(jnp.int32, (_K, _NB), 1) == flat_c
            dnr = jnp.sum(jnp.where(colmask, dn_ref[...], 0.0), axis=1,
                          keepdims=True)
            eir = jnp.sum(jnp.where(colmask, ei_ref[...], 0), axis=1,
                          keepdims=True)
            match = jnp.where(eir == b * _L + pos_o, 1.0, 0.0)
            mu = _MU0 + lax.broadcasted_iota(
                jnp.int32, (_K, 16), 1).astype(jnp.float32) * _MUSTEP
            t = (dnr - mu) * _INVSIG
            rb = jnp.exp(-(t * t))
            msum = jnp.sum(match * rb, axis=0, keepdims=True)
            edge = jnp.dot(msum, we_ref[...],
                           preferred_element_type=jnp.float32)
            g_seq = wtr - embed_of(mm_ref[b, i])
            fe_rows.append(jnp.concatenate([h1r, h2r, g_seq, edge], axis=1))
    fe = jnp.concatenate(fe_rows, axis=0)

    mean = jnp.mean(fe, axis=1, keepdims=True)
    cen = fe - mean
    var = jnp.mean(cen * cen, axis=1, keepdims=True)
    fe = cen * lax.rsqrt(var + 1e-5) * lng_ref[...] + lnb_ref[...]
    fe = jnp.maximum(jnp.dot(fe, wla_ref[...],
                             preferred_element_type=jnp.float32)
                     + bla_ref[...], 0.0)

    sm_rows = []
    for i in range(2):
        for b in range(_B):
            if i == 0:
                sm_rows.append(jnp.ones((1, 1), jnp.float32))
            else:
                dead = jnp.logical_and(mw_ref[b, 1] == 0, mm_ref[b, 1] == 0)
                sm_rows.append(jnp.where(dead, 0.0, 1.0) * jnp.ones((1, 1), jnp.float32))
    fe = fe * jnp.concatenate(sm_rows, axis=0)

    swapped = jnp.concatenate([fe[4:8], fe[0:4]], axis=0)
    embeds = jnp.concatenate([fe, swapped], axis=1)
    hmid = jnp.maximum(jnp.dot(embeds, wd1_ref[...],
                               preferred_element_type=jnp.float32)
                       + bd1_ref[...], 0.0)
    dd = jnp.dot(hmid, wd2_ref[...],
                 preferred_element_type=jnp.float32) + bd2_ref[...]

    lane441 = lax.broadcasted_iota(jnp.int32, (8, _VOCAB * _VOCAB), 1)
    mcols, wcols = [], []
    for i in range(2):
        for b in range(_B):
            if i == 0:
                mi = mm_ref[b, 0] * _VOCAB + mm_ref[b, 1]
                wi = mw_ref[b, 0] * _VOCAB + mw_ref[b, 1]
            else:
                mi = mm_ref[b, 1] * _VOCAB + mm_ref[b, 0]
                wi = mw_ref[b, 1] * _VOCAB + mw_ref[b, 0]
            mcols.append(jnp.full((1, 1), 0, jnp.int32) + mi)
            wcols.append(jnp.full((1, 1), 0, jnp.int32) + wi)
    mcol = jnp.concatenate(mcols, axis=0)
    wcol = jnp.concatenate(wcols, axis=0)
    mval = jnp.sum(jnp.where(lane441 == mcol, dd, 0.0), axis=1, keepdims=True)
    wval = jnp.sum(jnp.where(lane441 == wcol, dd, 0.0), axis=1, keepdims=True)
    diff = mval - wval
    res = 0.5 * (diff[0:4] + diff[4:8])
    res = jnp.concatenate([res, jnp.zeros((4, 1), jnp.float32)], axis=0)
    lane128 = lax.broadcasted_iota(jnp.int32, (8, _HID), 1)
    out_ref[...] = jnp.where(lane128 == 0, res, 0.0)


def _run_head(h1f, h2f, wtf, dnf, eif, mp, mw, mm, ws, we, lng, lnb, wla,
              bla, wd1, bd1, wd2, bd2):
    vspec = lambda shape: pl.BlockSpec(shape, lambda: tuple(0 for _ in shape))
    return pl.pallas_call(
        _head_body,
        in_specs=[
            vspec((_NB, _HID)), vspec((_NB, _HID)), vspec((_NB, _HID)),
            vspec((_K, _NB)), vspec((_K, _NB)),
            pl.BlockSpec(memory_space=pltpu.SMEM),
            pl.BlockSpec(memory_space=pltpu.SMEM),
            pl.BlockSpec(memory_space=pltpu.SMEM),
            vspec((_VOCAB, _HID)), vspec((16, _HID)),
            vspec((1, 512)), vspec((1, 512)),
            vspec((512, _HID)), vspec((1, _HID)),
            vspec((2 * _HID, _HID)), vspec((1, _HID)),
            vspec((_HID, _VOCAB * _VOCAB)), vspec((1, _VOCAB * _VOCAB)),
        ],
        out_specs=vspec((8, _HID)),
        out_shape=jax.ShapeDtypeStruct((8, _HID), jnp.float32),
    )(h1f, h2f, wtf, dnf, eif, mp, mw, mm, ws, we, lng, lnb, wla, bla,
      wd1, bd1, wd2, bd2)


def kernel(X, S, mask, chain_M, residue_idx, chain_encoding_all,
           mut_positions, mut_wildtype_AAs, mut_mutant_AAs, mut_ddGs,
           atom_mask, W_s, W_e, Wm1, bm1, Wm2, bm2, ln_g, ln_b, W_la, b_la,
           W_d1, b_d1, W_d2, b_d2):
    ca = X[:, :, 1, :]
    cat = jnp.transpose(ca, (0, 2, 1))
    s3 = S.astype(jnp.int32).reshape(_B, 1, _L)
    mp = mut_positions.astype(jnp.int32)
    mw = mut_wildtype_AAs.astype(jnp.int32)
    mm = mut_mutant_AAs.astype(jnp.int32)

    dn, ei, wt, g0, a0 = _run_prep(ca, cat, s3, mp, mw, W_s, Wm1, bm1)

    dn_f = dn.reshape(_K, _NB)
    ei_flat = ei.reshape(_ROWS)
    a_f = a0.reshape(_NB, _HID)
    h_f = wt.reshape(_NB, _HID)
    g_f = g0.reshape(_NB, _HID)

    gath0 = _gather_rows(g_f, ei_flat).reshape(_K, _NB, _HID)
    h1, g1, a1 = _run_comb(0, False, gath0, dn_f, a_f, h_f, Wm1, bm1,
                           Wm2, bm2, W_e)
    gath1 = _gather_rows(g1, ei_flat).reshape(_K, _NB, _HID)
    (h2,) = _run_comb(1, True, gath1, dn_f, a1, h1, Wm1, bm1, Wm2, bm2, W_e)

    res = _run_head(h1, h2, h_f, dn_f, ei.reshape(_K, _NB), mp, mw, mm,
                    W_s, W_e, ln_g.reshape(1, 512), ln_b.reshape(1, 512),
                    W_la, b_la.reshape(1, _HID), W_d1, b_d1.reshape(1, _HID),
                    W_d2, b_d2.reshape(1, _VOCAB * _VOCAB))
    return res[0:4, 0:1]
